# baseline (speedup 1.0000x reference)
import jax
import jax.numpy as jnp
from jax import lax
from jax.experimental import pallas as pl
from jax.experimental.pallas import tpu as pltpu
from jax.experimental.pallas import tpu_sc as plsc

B, S, D = 4, 8192, 768
NC, NS = 2, 16
NW = NC * NS
SPW = S // NW
SCHUNK = 64
NSC = SPW // SCHUNK
CW = SCHUNK * D
LANES = 16


def _sc_body(x_hbm, pos_hbm, out_hbm, posbuf, xbuf):
    wid = lax.axis_index("s") * NC + lax.axis_index("c")
    s0 = wid * SPW
    for sc in range(NSC):
        pltpu.sync_copy(pos_hbm.at[pl.ds((s0 + sc * SCHUNK) * D, CW)], posbuf)
        for b in range(B):
            off = (b * S + s0 + sc * SCHUNK) * D
            pltpu.sync_copy(x_hbm.at[pl.ds(off, CW)], xbuf)

            @plsc.parallel_loop(0, CW, step=LANES, unroll=8)
            def _(i):
                plsc.addupdate(xbuf.at[pl.ds(i, LANES)],
                               posbuf[pl.ds(i, LANES)])

            pltpu.sync_copy(xbuf, out_hbm.at[pl.ds(off, CW)])


_sc_call = pl.kernel(
    _sc_body,
    out_type=jax.ShapeDtypeStruct((B * S * D,), jnp.float32),
    mesh=plsc.VectorSubcoreMesh(core_axis_name="c", subcore_axis_name="s"),
    scratch_types=[
        pltpu.VMEM((CW,), jnp.float32),
        pltpu.VMEM((CW,), jnp.float32),
    ],
)


def kernel(x, pos_table):
    out = _sc_call(x.reshape(-1), pos_table.reshape(-1))
    return out.reshape(B, S, D)


# speedup vs baseline: 1.1571x; 1.1571x over previous
import jax
import jax.numpy as jnp
from jax import lax
from jax.experimental import pallas as pl
from jax.experimental.pallas import tpu as pltpu
from jax.experimental.pallas import tpu_sc as plsc

B, S, D = 4, 8192, 768
NC, NS = 2, 16
NW = NC * NS
SPW = S // NW
SCHUNK = 32
NSC = SPW // SCHUNK
CW = SCHUNK * D
NTASK = NSC * B
LANES = 16


def _sc_body(x_hbm, pos_hbm, out_hbm, pos0, pos1, xb0, xb1,
             sp0, sp1, si0, si1, so0, so1):
    wid = lax.axis_index("s") * NC + lax.axis_index("c")
    s0 = wid * SPW
    posb = (pos0, pos1)
    xb = (xb0, xb1)
    sp = (sp0, sp1)
    si = (si0, si1)
    so = (so0, so1)

    def xoff(t):
        sc, b = divmod(t, B)
        return (b * S + s0 + sc * SCHUNK) * D

    def pos_slice(sc):
        return pos_hbm.at[pl.ds((s0 + sc * SCHUNK) * D, CW)]

    pin = [pltpu.async_copy(pos_slice(0), pos0, sp0),
           pltpu.async_copy(pos_slice(1), pos1, sp1)]
    xin = [pltpu.async_copy(x_hbm.at[pl.ds(xoff(0), CW)], xb0, si0), None]
    xout = [None, None]

    for t in range(NTASK):
        sc, b = divmod(t, B)
        p = t & 1
        pc = sc & 1
        if b == 0:
            pin[pc].wait()
        xin[p].wait()
        if t + 1 < NTASK:
            q = (t + 1) & 1
            if xout[q] is not None:
                xout[q].wait()
            xin[q] = pltpu.async_copy(
                x_hbm.at[pl.ds(xoff(t + 1), CW)], xb[q], si[q])

        @plsc.parallel_loop(0, CW, step=LANES, unroll=8)
        def _(i):
            plsc.addupdate(xb[p].at[pl.ds(i, LANES)],
                           posb[pc][pl.ds(i, LANES)])

        if b == B - 1 and sc + 2 < NSC:
            pin[pc] = pltpu.async_copy(pos_slice(sc + 2), posb[pc], sp[pc])
        xout[p] = pltpu.async_copy(xb[p], out_hbm.at[pl.ds(xoff(t), CW)],
                                   so[p])
    xout[0].wait()
    xout[1].wait()


_sc_call = pl.kernel(
    _sc_body,
    out_type=jax.ShapeDtypeStruct((B * S * D,), jnp.float32),
    mesh=plsc.VectorSubcoreMesh(core_axis_name="c", subcore_axis_name="s"),
    scratch_types=[
        pltpu.VMEM((CW,), jnp.float32),
        pltpu.VMEM((CW,), jnp.float32),
        pltpu.VMEM((CW,), jnp.float32),
        pltpu.VMEM((CW,), jnp.float32),
        pltpu.SemaphoreType.DMA,
        pltpu.SemaphoreType.DMA,
        pltpu.SemaphoreType.DMA,
        pltpu.SemaphoreType.DMA,
        pltpu.SemaphoreType.DMA,
        pltpu.SemaphoreType.DMA,
    ],
)


def kernel(x, pos_table):
    out = _sc_call(x.reshape(-1), pos_table.reshape(-1))
    return out.reshape(B, S, D)


# speedup vs baseline: 3.1357x; 2.7100x over previous
import jax
import jax.numpy as jnp
from jax import lax
from jax.experimental import pallas as pl
from jax.experimental.pallas import tpu as pltpu
from jax.experimental.pallas import tpu_sc as plsc

B, S, D = 4, 8192, 768
NC, NS = 2, 16
NW = NC * NS
SPW = S // NW
SCHUNK = 32
NSC = SPW // SCHUNK
NTASK = NSC * B
LANES = 16
ROWS = B * S


def _sc_body(x_hbm, pos_hbm, out_hbm, pos0, pos1, xb0, xb1,
             sp0, sp1, si0, si1, so0, so1):
    wid = lax.axis_index("s") * NC + lax.axis_index("c")
    s0 = wid * SPW
    posb = (pos0, pos1)
    xb = (xb0, xb1)
    sp = (sp0, sp1)
    si = (si0, si1)
    so = (so0, so1)

    def xrow(t):
        sc, b = divmod(t, B)
        return b * S + s0 + sc * SCHUNK

    def pos_slice(sc):
        return pos_hbm.at[pl.ds(s0 + sc * SCHUNK, SCHUNK)]

    pin = [pltpu.async_copy(pos_slice(0), pos0, sp0),
           pltpu.async_copy(pos_slice(1), pos1, sp1)]
    xin = [pltpu.async_copy(x_hbm.at[pl.ds(xrow(0), SCHUNK)], xb0, si0), None]
    xout = [None, None]

    for t in range(NTASK):
        sc, b = divmod(t, B)
        p = t & 1
        pc = sc & 1
        if b == 0:
            pin[pc].wait()
        xin[p].wait()
        if t + 1 < NTASK:
            q = (t + 1) & 1
            if xout[q] is not None:
                xout[q].wait()
            xin[q] = pltpu.async_copy(
                x_hbm.at[pl.ds(xrow(t + 1), SCHUNK)], xb[q], si[q])

        @plsc.parallel_loop(0, SCHUNK, step=1)
        def _(r):
            @plsc.parallel_loop(0, D, step=LANES, unroll=8)
            def _(c):
                plsc.addupdate(xb[p].at[r].at[pl.ds(c, LANES)],
                               posb[pc].at[r][pl.ds(c, LANES)])

        if b == B - 1 and sc + 2 < NSC:
            pin[pc] = pltpu.async_copy(pos_slice(sc + 2), posb[pc], sp[pc])
        xout[p] = pltpu.async_copy(xb[p], out_hbm.at[pl.ds(xrow(t), SCHUNK)],
                                   so[p])
    xout[0].wait()
    xout[1].wait()


_sc_call = pl.kernel(
    _sc_body,
    out_type=jax.ShapeDtypeStruct((ROWS, D), jnp.float32),
    mesh=plsc.VectorSubcoreMesh(core_axis_name="c", subcore_axis_name="s"),
    scratch_types=[
        pltpu.VMEM((SCHUNK, D), jnp.float32),
        pltpu.VMEM((SCHUNK, D), jnp.float32),
        pltpu.VMEM((SCHUNK, D), jnp.float32),
        pltpu.VMEM((SCHUNK, D), jnp.float32),
        pltpu.SemaphoreType.DMA,
        pltpu.SemaphoreType.DMA,
        pltpu.SemaphoreType.DMA,
        pltpu.SemaphoreType.DMA,
        pltpu.SemaphoreType.DMA,
        pltpu.SemaphoreType.DMA,
    ],
)


def kernel(x, pos_table):
    out = _sc_call(x.reshape(ROWS, D), pos_table)
    return out.reshape(B, S, D)


# speedup vs baseline: 3.5222x; 1.1233x over previous
import jax
import jax.numpy as jnp
from jax import lax
from jax.experimental import pallas as pl
from jax.experimental.pallas import tpu as pltpu
from jax.experimental.pallas import tpu_sc as plsc

B, S, D = 4, 8192, 768
NC, NS = 2, 16
NW = NC * NS
SPW = S // NW
SCHUNK = 32
NSC = SPW // SCHUNK
NTASK = NSC * B
LANES = 16
NBUF = 4


def _sc_body(x_hbm, pos_hbm, out_hbm, posbuf, xb0, xb1, xb2, xb3,
             sp, si0, si1, si2, si3, so0, so1, so2, so3):
    wid = lax.axis_index("s") * NC + lax.axis_index("c")
    s0 = wid * SPW
    xb = (xb0, xb1, xb2, xb3)
    si = (si0, si1, si2, si3)
    so = (so0, so1, so2, so3)

    def xrow(t):
        sc, b = divmod(t, B)
        return b * S + s0 + sc * SCHUNK

    def pos_slice(sc):
        return pos_hbm.at[pl.ds(s0 + sc * SCHUNK, SCHUNK)]

    pin = pltpu.async_copy(pos_slice(0), posbuf, sp)
    xin = [None] * NBUF
    xout = [None] * NBUF
    xin[0] = pltpu.async_copy(x_hbm.at[pl.ds(xrow(0), SCHUNK)], xb[0], si[0])
    xin[1] = pltpu.async_copy(x_hbm.at[pl.ds(xrow(1), SCHUNK)], xb[1], si[1])

    for t in range(NTASK):
        sc, b = divmod(t, B)
        p = t % NBUF
        if b == 0:
            pin.wait()
        xin[p].wait()
        if t >= 1:
            q = (t - 1) % NBUF
            xout[q] = pltpu.async_copy(
                xb[q], out_hbm.at[pl.ds(xrow(t - 1), SCHUNK)], so[q])
        if t + 2 < NTASK:
            r = (t + 2) % NBUF
            if xout[r] is not None:
                xout[r].wait()
            xin[r] = pltpu.async_copy(
                x_hbm.at[pl.ds(xrow(t + 2), SCHUNK)], xb[r], si[r])

        @plsc.parallel_loop(0, SCHUNK, step=1)
        def _(rr):
            @plsc.parallel_loop(0, D, step=LANES, unroll=8)
            def _(c):
                plsc.addupdate(xb[p].at[rr].at[pl.ds(c, LANES)],
                               posbuf.at[rr][pl.ds(c, LANES)])

        if b == B - 1 and sc + 1 < NSC:
            pin = pltpu.async_copy(pos_slice(sc + 1), posbuf, sp)

    last = (NTASK - 1) % NBUF
    xout[last] = pltpu.async_copy(
        xb[last], out_hbm.at[pl.ds(xrow(NTASK - 1), SCHUNK)], so[last])
    for q in range(NBUF):
        if xout[q] is not None:
            xout[q].wait()


_sc_call = pl.kernel(
    _sc_body,
    out_type=jax.ShapeDtypeStruct((B * S, D), jnp.float32),
    mesh=plsc.VectorSubcoreMesh(core_axis_name="c", subcore_axis_name="s"),
    scratch_types=[
        pltpu.VMEM((SCHUNK, D), jnp.float32),
        pltpu.VMEM((SCHUNK, D), jnp.float32),
        pltpu.VMEM((SCHUNK, D), jnp.float32),
        pltpu.VMEM((SCHUNK, D), jnp.float32),
        pltpu.VMEM((SCHUNK, D), jnp.float32),
        pltpu.SemaphoreType.DMA,
        pltpu.SemaphoreType.DMA,
        pltpu.SemaphoreType.DMA,
        pltpu.SemaphoreType.DMA,
        pltpu.SemaphoreType.DMA,
        pltpu.SemaphoreType.DMA,
        pltpu.SemaphoreType.DMA,
        pltpu.SemaphoreType.DMA,
        pltpu.SemaphoreType.DMA,
    ],
)


def kernel(x, pos_table):
    out = _sc_call(x.reshape(B * S, D), pos_table)
    return out.reshape(B, S, D)
